# baseline (speedup 1.0000x reference)
import functools

import jax
import jax.numpy as jnp
from jax import lax
from jax.experimental import pallas as pl
from jax.experimental.pallas import tpu as pltpu
from jax.experimental.pallas import tpu_sc as plsc

_NUM_CLASSES = 1000
_D = 128
_B = 16384
_NC = 1
_NS = 16
_NW = _NC * _NS
_BPW = _B // _NW
_CH = 128
_NCHUNK = _BPW // _CH
_VPR = _D // 16
_UNROLL = 4


def _sc_body(feats_hbm, labels_hbm, centers_hbm, out_hbm,
             idx_v, rows0, rows1, feats0, feats1, acc_v,
             gsem0, gsem1, fsem0, fsem1):
    rows_v = (rows0, rows1)
    feats_v = (feats0, feats1)
    gsem = (gsem0, gsem1)
    fsem = (fsem0, fsem1)

    wid = lax.axis_index("s") * _NC + lax.axis_index("c")
    base = wid * _BPW
    pltpu.sync_copy(labels_hbm.at[pl.ds(wid * _NCHUNK, _NCHUNK)], idx_v)

    def start(c):
        buf = c % 2
        hg = pltpu.async_copy(centers_hbm.at[idx_v.at[c]], rows_v[buf], gsem[buf])
        hf = pltpu.async_copy(feats_hbm.at[pl.ds(base + c * _CH, _CH)],
                              feats_v[buf], fsem[buf])
        return hg, hf

    pending = [start(0), start(1)]
    acc = tuple(jnp.zeros((16,), jnp.float32) for _ in range(_VPR))
    for c in range(_NCHUNK):
        buf = c % 2
        hg, hf = pending[c]
        hg.wait()
        hf.wait()

        fv = feats_v[buf]
        rv = rows_v[buf]

        def body(it, acc, fv=fv, rv=rv):
            i = it * _UNROLL
            out = list(acc)
            for r in range(_UNROLL):
                for j in range(_VPR):
                    f = fv[i + r, pl.ds(j * 16, 16)]
                    ctr = rv[i + r, pl.ds(j * 16, 16)]
                    d = f - ctr
                    out[j] = out[j] + d * d
            return tuple(out)

        acc = lax.fori_loop(0, _CH // _UNROLL, body, acc)
        if c + 2 < _NCHUNK:
            pending.append(start(c + 2))

    total = acc[0]
    for j in range(1, _VPR):
        total = total + acc[j]
    acc_v[...] = total
    pltpu.sync_copy(acc_v, out_hbm.at[wid])


@jax.jit
def kernel(feats, labels, centers_weight):
    labels2d = jnp.squeeze(labels).astype(jnp.int32).reshape(_B // _CH, _CH)
    mesh = plsc.VectorSubcoreMesh(core_axis_name="c", subcore_axis_name="s", num_cores=1)
    partial_fn = functools.partial(
        pl.kernel,
        mesh=mesh,
        out_type=jax.ShapeDtypeStruct((_NW, 16), jnp.float32),
        scratch_types=[
            pltpu.VMEM((_NCHUNK, _CH), jnp.int32),
            pltpu.VMEM((_CH, _D), jnp.float32),
            pltpu.VMEM((_CH, _D), jnp.float32),
            pltpu.VMEM((_CH, _D), jnp.float32),
            pltpu.VMEM((_CH, _D), jnp.float32),
            pltpu.VMEM((16,), jnp.float32),
            pltpu.SemaphoreType.DMA,
            pltpu.SemaphoreType.DMA,
            pltpu.SemaphoreType.DMA,
            pltpu.SemaphoreType.DMA,
        ],
    )(_sc_body)
    partials = partial_fn(feats, labels2d, centers_weight)
    return jnp.sum(partials) / jnp.float32(_B * _D)


# speedup vs baseline: 1.1799x; 1.1799x over previous
import functools

import jax
import jax.numpy as jnp
from jax import lax
from jax.experimental import pallas as pl
from jax.experimental.pallas import tpu as pltpu
from jax.experimental.pallas import tpu_sc as plsc

_K = 1000
_KP = 1024
_D = 128
_B = 16384
_NC = 2
_NS = 16
_NW = _NC * _NS
_BPW = _B // _NW
_CH = 128
_NCHUNK = _BPW // _CH
_VPR = _D // 16
_CROWS = _KP // _NS


def _sc_body(feats_hbm, labels_hbm, cpad_hbm, q_hbm, out_hbm,
             idx_v, f0, f1, q_v, cbuf, sbuf, acc_v,
             s_shared, fsem0, fsem1, qsem):
    fbuf = (f0, f1)
    fsem = (fsem0, fsem1)
    sid = lax.axis_index("s")
    wid = sid * _NC + lax.axis_index("c")

    pltpu.sync_copy(labels_hbm.at[pl.ds(wid * _NCHUNK, _NCHUNK)], idx_v)
    qcopy = pltpu.async_copy(q_hbm, q_v, qsem)

    def start(c):
        return pltpu.async_copy(
            feats_hbm.at[pl.ds(wid * _BPW + c * _CH, _CH)], fbuf[c % 2],
            fsem[c % 2])

    pending = [start(0), start(1)]

    def zfill(i, _):
        for j in range(_VPR):
            cbuf[i, pl.ds(j * 16, 16)] = jnp.zeros((16,), jnp.float32)
        return 0

    lax.fori_loop(0, _CROWS, zfill, 0)
    pltpu.sync_copy(cbuf, s_shared.at[pl.ds(sid * _CROWS, _CROWS)])
    plsc.subcore_barrier()

    for c in range(_NCHUNK):
        pending[c].wait()
        pltpu.sync_copy(fbuf[c % 2], s_shared.at[idx_v.at[c]], add=True)
        if c + 2 < _NCHUNK:
            pending.append(start(c + 2))

    qcopy.wait()
    acc = tuple(jnp.zeros((16,), jnp.float32) for _ in range(_VPR))

    def qbody(t, acc):
        out = list(acc)
        for g in range(_VPR):
            lab = idx_v[t, pl.ds(g * 16, 16)]
            out[g] = out[g] + plsc.load_gather(q_v, [lab])
        return tuple(out)

    acc = lax.fori_loop(0, _NCHUNK, qbody, acc)
    plsc.subcore_barrier()

    base = sid * _CROWS
    pltpu.sync_copy(cpad_hbm.at[pl.ds(base, _CROWS)], cbuf)
    pltpu.sync_copy(s_shared.at[pl.ds(base, _CROWS)], sbuf)

    def body(i, acc):
        out = list(acc)
        for j in range(_VPR):
            cc = cbuf[i, pl.ds(j * 16, 16)]
            ss = sbuf[i, pl.ds(j * 16, 16)]
            out[j] = out[j] - (cc + cc) * ss
        return tuple(out)

    acc = lax.fori_loop(0, _CROWS, body, acc)
    total = acc[0]
    for j in range(1, _VPR):
        total = total + acc[j]
    acc_v[...] = total
    pltpu.sync_copy(acc_v, out_hbm.at[wid])


def _tc_prep_body(c_ref, cpad_ref, q_ref):
    x = c_ref[...]
    zpad = jnp.zeros((_KP - _K, _D), jnp.float32)
    cp = jnp.concatenate([x, zpad], axis=0)
    cpad_ref[...] = cp
    q_ref[...] = jnp.sum(cp.reshape(_KP // _D, _D, _D) ** 2, axis=2)


def _tc_sumsq_body(x_ref, o_ref):
    @pl.when(pl.program_id(0) == 0)
    def _():
        o_ref[0, 0] = 0.0

    x = x_ref[...]
    o_ref[0, 0] += jnp.sum(x * x)


@jax.jit
def kernel(feats, labels, centers_weight):
    labels2d = jnp.squeeze(labels).astype(jnp.int32).reshape(_B // _CH, _CH)

    cpad, q2d = pl.pallas_call(
        _tc_prep_body,
        out_shape=(jax.ShapeDtypeStruct((_KP, _D), jnp.float32),
                   jax.ShapeDtypeStruct((_KP // _D, _D), jnp.float32)),
    )(centers_weight)

    mesh = plsc.VectorSubcoreMesh(core_axis_name="c", subcore_axis_name="s")
    sc_fn = functools.partial(
        pl.kernel,
        mesh=mesh,
        out_type=jax.ShapeDtypeStruct((_NW, 16), jnp.float32),
        compiler_params=pltpu.CompilerParams(needs_layout_passes=False),
        scratch_types=[
            pltpu.VMEM((_NCHUNK, _CH), jnp.int32),
            pltpu.VMEM((_CH, _D), jnp.float32),
            pltpu.VMEM((_CH, _D), jnp.float32),
            pltpu.VMEM((_KP,), jnp.float32),
            pltpu.VMEM((_CROWS, _D), jnp.float32),
            pltpu.VMEM((_CROWS, _D), jnp.float32),
            pltpu.VMEM((16,), jnp.float32),
            pltpu.VMEM_SHARED((_KP, _D), jnp.float32),
            pltpu.SemaphoreType.DMA,
            pltpu.SemaphoreType.DMA,
            pltpu.SemaphoreType.DMA,
        ],
    )(_sc_body)
    partials = sc_fn(feats, labels2d, cpad, q2d.reshape(_KP))

    sumsq = pl.pallas_call(
        _tc_sumsq_body,
        grid=(16,),
        in_specs=[pl.BlockSpec((_B // 16, _D), lambda i: (i, 0))],
        out_specs=pl.BlockSpec(memory_space=pltpu.SMEM),
        out_shape=jax.ShapeDtypeStruct((1, 1), jnp.float32),
    )(feats)

    return (sumsq[0, 0] + jnp.sum(partials)) / jnp.float32(_B * _D)


# speedup vs baseline: 1.2303x; 1.0427x over previous
import functools

import jax
import jax.numpy as jnp
from jax import lax
from jax.experimental import pallas as pl
from jax.experimental.pallas import tpu as pltpu
from jax.experimental.pallas import tpu_sc as plsc

_K = 1000
_KP = 1024
_D = 128
_B = 16384
_NC = 2
_NS = 16
_NW = _NC * _NS
_BPW = _B // _NW
_CH = 128
_NCHUNK = _BPW // _CH
_VPR = _D // 16
_CROWS = 64
_CROWS_LAST = _K - 15 * _CROWS


def _sc_body(feats_hbm, labels_hbm, centers_hbm, out_hbm,
             idx_v, f0, f1, f2, f3, hist_v, ridx_v, cbuf, sbuf, nbuf, acc_v,
             s_shared, n_shared,
             fs0, fs1, fs2, fs3, ss0, ss1, ss2, ss3, csem):
    fbuf = (f0, f1, f2, f3)
    fsem = (fs0, fs1, fs2, fs3)
    ssem = (ss0, ss1, ss2, ss3)
    sid = lax.axis_index("s")
    wid = sid * _NC + lax.axis_index("c")

    pltpu.sync_copy(labels_hbm.at[pl.ds(wid * _NCHUNK, _NCHUNK)], idx_v)
    fpend = [
        pltpu.async_copy(
            feats_hbm.at[pl.ds(wid * _BPW + c * _CH, _CH)], fbuf[c], fsem[c])
        for c in range(_NCHUNK)
    ]
    cpend = pltpu.async_copy(
        centers_hbm.at[pl.ds(sid * _CROWS, _CROWS)], cbuf, csem)

    for r in range(2 * _KP // _D):
        def hzero(i, _, r=r):
            hist_v[r, pl.ds(i * 16, 16)] = jnp.zeros((16,), jnp.float32)
            return 0

        lax.fori_loop(0, _D // 16, hzero, 0)

    ones = jnp.ones((16,), jnp.float32)
    for c in range(_NCHUNK):
        def hadd(g, _, c=c):
            lab = idx_v[c, pl.ds(g * 16, 16)]
            row = lax.shift_right_logical(lab, 7)
            col = lax.bitwise_and(lab, 127)
            plsc.addupdate_scatter(hist_v, [row, col], ones)
            return 0

        lax.fori_loop(0, _VPR, hadd, 0)

    ridx_v[...] = lax.iota(jnp.int32, 16)

    def zfill(i, _):
        for j in range(_VPR):
            sbuf[i, pl.ds(j * 16, 16)] = jnp.zeros((16,), jnp.float32)
        return 0

    lax.fori_loop(0, _CROWS, zfill, 0)
    pltpu.sync_copy(sbuf, s_shared.at[pl.ds(sid * _CROWS, _CROWS)])

    @pl.when(sid == 0)
    def _():
        pltpu.sync_copy(sbuf.at[pl.ds(0, 2 * _KP // _D)], n_shared)

    plsc.subcore_barrier()

    spend = []
    for c in range(_NCHUNK):
        fpend[c].wait()
        spend.append(pltpu.async_copy(
            fbuf[c], s_shared.at[idx_v.at[c]], ssem[c], add=True))
    pltpu.sync_copy(hist_v, n_shared.at[ridx_v], add=True)
    for c in range(_NCHUNK):
        spend[c].wait()
    plsc.subcore_barrier()

    acc = tuple(jnp.zeros((16,), jnp.float32) for _ in range(_VPR))

    def combine(nrows):
        base = sid * _CROWS
        pltpu.sync_copy(s_shared.at[pl.ds(base, nrows)],
                        sbuf.at[pl.ds(0, nrows)])
        pltpu.sync_copy(n_shared, nbuf)

        def body(i, acc):
            out = list(acc)
            c = base + i
            rowv = jnp.full((16,), lax.shift_right_logical(c, 7), jnp.int32)
            colv = jnp.full((16,), lax.bitwise_and(c, 127), jnp.int32)
            n = plsc.load_gather(nbuf, [rowv, colv])
            for j in range(_VPR):
                cc = cbuf[i, pl.ds(j * 16, 16)]
                ss = sbuf[i, pl.ds(j * 16, 16)]
                t = n * cc - (ss + ss)
                out[j] = out[j] + cc * t
            return tuple(out)

        return lax.fori_loop(0, nrows, body, acc)

    cpend.wait()

    @pl.when(sid < 15)
    def _():
        acc2 = combine(_CROWS)
        total = acc2[0]
        for j in range(1, _VPR):
            total = total + acc2[j]
        acc_v[...] = total

    @pl.when(sid == 15)
    def _():
        acc2 = combine(_CROWS_LAST)
        total = acc2[0]
        for j in range(1, _VPR):
            total = total + acc2[j]
        acc_v[...] = total

    pltpu.sync_copy(acc_v, out_hbm.at[wid])


def _tc_sumsq_body(x_ref, o_ref, acc_ref):
    @pl.when(pl.program_id(0) == 0)
    def _():
        acc_ref[...] = jnp.zeros((8, _D), jnp.float32)

    x = x_ref[...]
    xx = x * x
    acc_ref[...] += jnp.sum(xx.reshape(_B // 16 // 8, 8, _D), axis=0)

    @pl.when(pl.program_id(0) == 15)
    def _():
        o_ref[0, 0] = jnp.sum(acc_ref[...])


@jax.jit
def kernel(feats, labels, centers_weight):
    labels2d = jnp.squeeze(labels).astype(jnp.int32).reshape(_B // _CH, _CH)
    mesh = plsc.VectorSubcoreMesh(core_axis_name="c", subcore_axis_name="s")
    sc_fn = functools.partial(
        pl.kernel,
        mesh=mesh,
        out_type=jax.ShapeDtypeStruct((_NW, 16), jnp.float32),
        compiler_params=pltpu.CompilerParams(needs_layout_passes=False),
        scratch_types=[
            pltpu.VMEM((_NCHUNK, _CH), jnp.int32),
            pltpu.VMEM((_CH, _D), jnp.float32),
            pltpu.VMEM((_CH, _D), jnp.float32),
            pltpu.VMEM((_CH, _D), jnp.float32),
            pltpu.VMEM((_CH, _D), jnp.float32),
            pltpu.VMEM((2 * _KP // _D, _D), jnp.float32),
            pltpu.VMEM((16,), jnp.int32),
            pltpu.VMEM((_CROWS, _D), jnp.float32),
            pltpu.VMEM((_CROWS, _D), jnp.float32),
            pltpu.VMEM((2 * _KP // _D, _D), jnp.float32),
            pltpu.VMEM((16,), jnp.float32),
            pltpu.VMEM_SHARED((_KP, _D), jnp.float32),
            pltpu.VMEM_SHARED((2 * _KP // _D, _D), jnp.float32),
            pltpu.SemaphoreType.DMA,
            pltpu.SemaphoreType.DMA,
            pltpu.SemaphoreType.DMA,
            pltpu.SemaphoreType.DMA,
            pltpu.SemaphoreType.DMA,
            pltpu.SemaphoreType.DMA,
            pltpu.SemaphoreType.DMA,
            pltpu.SemaphoreType.DMA,
            pltpu.SemaphoreType.DMA,
        ],
    )(_sc_body)
    partials = sc_fn(feats, labels2d, centers_weight)

    sumsq = pl.pallas_call(
        _tc_sumsq_body,
        grid=(16,),
        in_specs=[pl.BlockSpec((_B // 16, _D), lambda i: (i, 0))],
        out_specs=pl.BlockSpec(memory_space=pltpu.SMEM),
        out_shape=jax.ShapeDtypeStruct((1, 1), jnp.float32),
        scratch_shapes=[pltpu.VMEM((8, _D), jnp.float32)],
    )(feats)

    return (sumsq[0, 0] + jnp.sum(partials)) / jnp.float32(_B * _D)
